# baseline (speedup 1.0000x reference)
import functools

import jax
import jax.numpy as jnp
from jax.experimental import pallas as pl
from jax.experimental.pallas import tpu as pltpu

B = 8192
D = 2048
E = 64
TOP_K = 2
Z_LOSS_WEIGHT = 1e-4
BLK = 1024
NSTEP = B // BLK


def _gate_kernel(x_ref, wt_ref, b_ref,
                 i1_ref, i2_ref, s1_ref, s2_ref, imp_ref, z_ref):
    x = x_ref[...]
    logits = jnp.dot(x, wt_ref[...], preferred_element_type=jnp.float32)
    logits = logits + b_ref[...]

    m1 = jnp.max(logits, axis=-1, keepdims=True)
    ex = jnp.exp(logits - m1)
    ssum = jnp.sum(ex, axis=-1, keepdims=True)
    lse = m1 + jnp.log(ssum)

    iota = jax.lax.broadcasted_iota(
        jnp.int32, logits.shape, 1).astype(jnp.float32)
    i1 = jnp.min(jnp.where(logits == m1, iota, float(E)), axis=-1,
                 keepdims=True)
    masked = jnp.where(iota == i1, -jnp.inf, logits)
    m2 = jnp.max(masked, axis=-1, keepdims=True)
    i2 = jnp.min(jnp.where(masked == m2, iota, float(E)), axis=-1,
                 keepdims=True)

    p1 = jnp.exp(m1 - lse)
    p2 = jnp.exp(m2 - lse)
    rtot = 1.0 / (p1 + p2 + 1e-8)
    i1_ref[...] = i1[:, 0].astype(jnp.int32)
    i2_ref[...] = i2[:, 0].astype(jnp.int32)
    s1_ref[...] = (p1 * rtot)[:, 0]
    s2_ref[...] = (p2 * rtot)[:, 0]

    ones8 = jnp.ones((8, BLK), jnp.float32)
    probs = ex * (1.0 / ssum)
    imp_ref[0] = jnp.dot(ones8, probs, preferred_element_type=jnp.float32)
    z_ref[0] = jnp.dot(ones8, lse * lse, preferred_element_type=jnp.float32)


def _loss_kernel(imp_ref, z_ref, loss_ref):
    imp = jnp.sum(imp_ref[...], axis=(0, 1)) * (1.0 / (8 * B))
    imp = imp.reshape(1, E)
    mean = jnp.mean(imp, axis=-1, keepdims=True)
    var = jnp.mean((imp - mean) ** 2, axis=-1, keepdims=True)
    lb = var / (mean * mean + 1e-8)
    z = jnp.sum(z_ref[...], axis=(0, 1), keepdims=False)
    z = z.reshape(1, 1) * (Z_LOSS_WEIGHT / (8 * B))
    loss_ref[...] = lb + z


@functools.partial(jax.jit)
def _gate(gml, W, b):
    wt = W.T
    b2 = b.reshape(1, E)
    i1, i2, s1, s2, imp_p, z_p = pl.pallas_call(
        _gate_kernel,
        grid=(NSTEP,),
        in_specs=[
            pl.BlockSpec((BLK, D), lambda i: (i, 0)),
            pl.BlockSpec((D, E), lambda i: (0, 0)),
            pl.BlockSpec((1, E), lambda i: (0, 0)),
        ],
        out_specs=[
            pl.BlockSpec((BLK,), lambda i: (i,)),
            pl.BlockSpec((BLK,), lambda i: (i,)),
            pl.BlockSpec((BLK,), lambda i: (i,)),
            pl.BlockSpec((BLK,), lambda i: (i,)),
            pl.BlockSpec((1, 8, E), lambda i: (i, 0, 0)),
            pl.BlockSpec((1, 8, 1), lambda i: (i, 0, 0)),
        ],
        out_shape=[
            jax.ShapeDtypeStruct((B,), jnp.int32),
            jax.ShapeDtypeStruct((B,), jnp.int32),
            jax.ShapeDtypeStruct((B,), jnp.float32),
            jax.ShapeDtypeStruct((B,), jnp.float32),
            jax.ShapeDtypeStruct((NSTEP, 8, E), jnp.float32),
            jax.ShapeDtypeStruct((NSTEP, 8, 1), jnp.float32),
        ],
        compiler_params=pltpu.CompilerParams(
            dimension_semantics=("parallel",)),
    )(gml, wt, b2)
    loss = pl.pallas_call(
        _loss_kernel,
        out_shape=jax.ShapeDtypeStruct((1, 1), jnp.float32),
    )(imp_p, z_p)
    idx = jnp.stack([i1, i2], axis=-1)
    score = jnp.stack([s1, s2], axis=-1)
    return idx, score, loss[0, 0]


def kernel(gml, W, b):
    return _gate(gml, W, b)


# speedup vs baseline: 1.0276x; 1.0276x over previous
import functools

import jax
import jax.numpy as jnp
from jax.experimental import pallas as pl
from jax.experimental.pallas import tpu as pltpu

B = 8192
D = 2048
E = 64
TOP_K = 2
Z_LOSS_WEIGHT = 1e-4
BLK = 1024
DH = D // 2


def _gate_kernel(xl_ref, xr_ref, wtl_ref, wtr_ref, b_ref,
                 i1_ref, i2_ref, s1_ref, s2_ref, loss_ref,
                 imp_ref, z_ref):
    step = pl.program_id(0)

    @pl.when(step == 0)
    def _init():
        imp_ref[...] = jnp.zeros_like(imp_ref)
        z_ref[...] = jnp.zeros_like(z_ref)

    logits = jnp.dot(xl_ref[...], wtl_ref[...],
                     preferred_element_type=jnp.float32)
    logits += jnp.dot(xr_ref[...], wtr_ref[...],
                      preferred_element_type=jnp.float32)
    logits = logits + b_ref[...]

    m1 = jnp.max(logits, axis=-1, keepdims=True)
    ex = jnp.exp(logits - m1)
    ssum = jnp.sum(ex, axis=-1, keepdims=True)
    lse = m1 + jnp.log(ssum)

    iota = jax.lax.broadcasted_iota(
        jnp.int32, logits.shape, 1).astype(jnp.float32)
    i1 = jnp.min(jnp.where(logits == m1, iota, float(E)), axis=-1,
                 keepdims=True)
    masked = jnp.where(iota == i1, -jnp.inf, logits)
    m2 = jnp.max(masked, axis=-1, keepdims=True)
    i2 = jnp.min(jnp.where(masked == m2, iota, float(E)), axis=-1,
                 keepdims=True)

    p1 = jnp.exp(m1 - lse)
    p2 = jnp.exp(m2 - lse)
    rtot = 1.0 / (p1 + p2 + 1e-8)
    i1_ref[...] = i1[:, 0].astype(jnp.int32)
    i2_ref[...] = i2[:, 0].astype(jnp.int32)
    s1_ref[...] = (p1 * rtot)[:, 0]
    s2_ref[...] = (p2 * rtot)[:, 0]

    ones8 = jnp.ones((8, BLK), jnp.float32)
    probs = ex * (1.0 / ssum)
    imp_ref[...] += jnp.dot(ones8, probs,
                            preferred_element_type=jnp.float32)
    z_ref[...] += jnp.dot(ones8, lse * lse,
                          preferred_element_type=jnp.float32)

    @pl.when(step == pl.num_programs(0) - 1)
    def _finish():
        imp = jnp.sum(imp_ref[...], axis=0, keepdims=True) * (1.0 / (8 * B))
        mean = jnp.mean(imp, axis=-1, keepdims=True)
        var = jnp.mean((imp - mean) ** 2, axis=-1, keepdims=True)
        lb = var / (mean * mean + 1e-8)
        z = jnp.sum(z_ref[...], axis=0, keepdims=True) * (Z_LOSS_WEIGHT / (8 * B))
        loss_ref[...] = lb + z


@functools.partial(jax.jit)
def _gate(gml, W, b):
    wt = W.T
    b2 = b.reshape(1, E)
    out = pl.pallas_call(
        _gate_kernel,
        grid=(B // BLK,),
        in_specs=[
            pl.BlockSpec((BLK, DH), lambda i: (i, 0)),
            pl.BlockSpec((BLK, DH), lambda i: (i, 1)),
            pl.BlockSpec((DH, E), lambda i: (0, 0)),
            pl.BlockSpec((DH, E), lambda i: (1, 0)),
            pl.BlockSpec((1, E), lambda i: (0, 0)),
        ],
        out_specs=[
            pl.BlockSpec((BLK,), lambda i: (i,)),
            pl.BlockSpec((BLK,), lambda i: (i,)),
            pl.BlockSpec((BLK,), lambda i: (i,)),
            pl.BlockSpec((BLK,), lambda i: (i,)),
            pl.BlockSpec((1, 1), lambda i: (0, 0)),
        ],
        out_shape=[
            jax.ShapeDtypeStruct((B,), jnp.int32),
            jax.ShapeDtypeStruct((B,), jnp.int32),
            jax.ShapeDtypeStruct((B,), jnp.float32),
            jax.ShapeDtypeStruct((B,), jnp.float32),
            jax.ShapeDtypeStruct((1, 1), jnp.float32),
        ],
        scratch_shapes=[
            pltpu.VMEM((8, E), jnp.float32),
            pltpu.VMEM((8, 1), jnp.float32),
        ],
    )(gml, gml, wt, wt, b2)
    i1, i2, s1, s2, loss = out
    idx = jnp.stack([i1, i2], axis=-1)
    score = jnp.stack([s1, s2], axis=-1)
    return idx, score, loss[0, 0]


def kernel(gml, W, b):
    return _gate(gml, W, b)
